# baseline (speedup 1.0000x reference)
import functools

import jax
import jax.numpy as jnp
from jax.experimental import pallas as pl
from jax.experimental.pallas import tpu as pltpu
from jax.experimental.pallas import tpu_sc as plsc

_N = 256
_BB = 8
_NC = 2
_NS = 16
_BPW = _N // (_NC * _NS)
_L = 16


def _offs_body(rand_ref, offs_ref):
    r = rand_ref[...]
    n = r.shape[1]
    less = r[:, :, None] < r[:, None, :]
    kk = jax.lax.broadcasted_iota(jnp.int32, (1, n, n), 1)
    jj = jax.lax.broadcasted_iota(jnp.int32, (1, n, n), 2)
    tie = (r[:, :, None] == r[:, None, :]) & (kk < jj)
    rank = jnp.sum((less | tie).astype(jnp.int32), axis=1)
    j1 = jax.lax.broadcasted_iota(jnp.int32, (1, n), 1)
    offs_ref[...] = rank * n + j1


def _zeros_body(row_ref):
    row_ref[...] = jnp.zeros_like(row_ref)


def _sc_col_body(offs_hbm, zrow_hbm, col_hbm, buf, offs_v):
    cid = jax.lax.axis_index("c")
    sid = jax.lax.axis_index("s")
    base = (sid * _NC + cid) * _BPW
    pltpu.sync_copy(offs_hbm.at[pl.ds(base, _BPW)], offs_v)
    pltpu.sync_copy(zrow_hbm, buf)
    ones = jnp.full((_L,), 1.0, jnp.float32)
    zeros = jnp.zeros((_L,), jnp.float32)
    for bi in range(_BPW):
        for c in range(_N // _L):
            plsc.store_scatter(buf, [offs_v[bi, pl.ds(c * _L, _L)]], ones)
        pltpu.sync_copy(buf, col_hbm.at[base + bi])
        for c in range(_N // _L):
            plsc.store_scatter(buf, [offs_v[bi, pl.ds(c * _L, _L)]], zeros)


_sc_col = functools.partial(
    pl.kernel,
    out_type=jax.ShapeDtypeStruct((_N, _N * _N), jnp.float32),
    mesh=plsc.VectorSubcoreMesh(core_axis_name="c", subcore_axis_name="s"),
    scratch_types=[
        pltpu.VMEM((_N * _N,), jnp.float32),
        pltpu.VMEM((_BPW, _N), jnp.int32),
    ],
    compiler_params=pltpu.CompilerParams(needs_layout_passes=False),
)(_sc_col_body)


def kernel(cost_matrix):
    b, n, _ = cost_matrix.shape
    rkey = jax.random.fold_in(jax.random.key(0), 1)
    rand = jax.random.uniform(rkey, (b, n), dtype=jnp.float32)
    offs = pl.pallas_call(
        _offs_body,
        grid=(b // _BB,),
        in_specs=[pl.BlockSpec((_BB, n), lambda i: (i, 0))],
        out_specs=pl.BlockSpec((_BB, n), lambda i: (i, 0)),
        out_shape=jax.ShapeDtypeStruct((b, n), jnp.int32),
    )(rand)
    row_emb = pl.pallas_call(
        _zeros_body,
        grid=(b // _BB,),
        out_specs=pl.BlockSpec((_BB, n, n), lambda i: (i, 0, 0)),
        out_shape=jax.ShapeDtypeStruct((b, n, n), cost_matrix.dtype),
    )()
    zrow = jnp.zeros((n * n,), dtype=jnp.float32)
    col_emb = _sc_col(offs, zrow).reshape(b, n, n)
    return (row_emb, col_emb, cost_matrix)


# speedup vs baseline: 1.0080x; 1.0080x over previous
import functools

import jax
import jax.numpy as jnp
from jax.experimental import pallas as pl
from jax.experimental.pallas import tpu as pltpu
from jax.experimental.pallas import tpu_sc as plsc

_N = 256
_BB = 8
_NC = 2
_NS = 16
_BPW = _N // (_NC * _NS)
_L = 16


def _offs_body(rand_ref, offs_ref):
    r = rand_ref[...]
    n = r.shape[1]
    less = r[:, :, None] < r[:, None, :]
    kk = jax.lax.broadcasted_iota(jnp.int32, (1, n, n), 1)
    jj = jax.lax.broadcasted_iota(jnp.int32, (1, n, n), 2)
    tie = (r[:, :, None] == r[:, None, :]) & (kk < jj)
    rank = jnp.sum((less | tie).astype(jnp.int32), axis=1)
    j1 = jax.lax.broadcasted_iota(jnp.int32, (1, n), 1)
    offs_ref[...] = rank * n + j1


def _zeros_copy_body(cost_ref, row_ref, cost_out_ref):
    row_ref[...] = jnp.zeros_like(row_ref)
    cost_out_ref[...] = cost_ref[...]


def _sc_col_body(offs_hbm, zrow_hbm, col_hbm, buf, offs_v):
    cid = jax.lax.axis_index("c")
    sid = jax.lax.axis_index("s")
    base = (sid * _NC + cid) * _BPW
    pltpu.sync_copy(offs_hbm.at[pl.ds(base, _BPW)], offs_v)
    pltpu.sync_copy(zrow_hbm, buf)
    ones = jnp.full((_L,), 1.0, jnp.float32)
    zeros = jnp.zeros((_L,), jnp.float32)
    for bi in range(_BPW):
        for c in range(_N // _L):
            plsc.store_scatter(buf, [offs_v[bi, pl.ds(c * _L, _L)]], ones)
        pltpu.sync_copy(buf, col_hbm.at[base + bi])
        for c in range(_N // _L):
            plsc.store_scatter(buf, [offs_v[bi, pl.ds(c * _L, _L)]], zeros)


_sc_col = functools.partial(
    pl.kernel,
    out_type=jax.ShapeDtypeStruct((_N, _N * _N), jnp.float32),
    mesh=plsc.VectorSubcoreMesh(core_axis_name="c", subcore_axis_name="s"),
    scratch_types=[
        pltpu.VMEM((_N * _N,), jnp.float32),
        pltpu.VMEM((_BPW, _N), jnp.int32),
    ],
    compiler_params=pltpu.CompilerParams(needs_layout_passes=False),
)(_sc_col_body)


def kernel(cost_matrix):
    b, n, _ = cost_matrix.shape
    rkey = jax.random.fold_in(jax.random.key(0), 1)
    rand = jax.random.uniform(rkey, (b, n), dtype=jnp.float32)
    offs = pl.pallas_call(
        _offs_body,
        grid=(b // _BB,),
        in_specs=[pl.BlockSpec((_BB, n), lambda i: (i, 0))],
        out_specs=pl.BlockSpec((_BB, n), lambda i: (i, 0)),
        out_shape=jax.ShapeDtypeStruct((b, n), jnp.int32),
    )(rand)
    row_emb, cost_out = pl.pallas_call(
        _zeros_copy_body,
        grid=(b // _BB,),
        in_specs=[pl.BlockSpec((_BB, n, n), lambda i: (i, 0, 0))],
        out_specs=[
            pl.BlockSpec((_BB, n, n), lambda i: (i, 0, 0)),
            pl.BlockSpec((_BB, n, n), lambda i: (i, 0, 0)),
        ],
        out_shape=[
            jax.ShapeDtypeStruct((b, n, n), cost_matrix.dtype),
            jax.ShapeDtypeStruct((b, n, n), cost_matrix.dtype),
        ],
    )(cost_matrix)
    zrow = jnp.zeros((n * n,), dtype=jnp.float32)
    col_emb = _sc_col(offs, zrow).reshape(b, n, n)
    return (row_emb, col_emb, cost_out)


# speedup vs baseline: 1.3778x; 1.3669x over previous
import functools

import jax
import jax.numpy as jnp
from jax.experimental import pallas as pl
from jax.experimental.pallas import tpu as pltpu
from jax.experimental.pallas import tpu_sc as plsc

_N = 256
_BB = 8
_NC = 2
_NS = 16
_BPW = _N // (_NC * _NS)
_L = 16


def _rank_body(rand_ref, rank_ref):
    r = rand_ref[...]
    n = r.shape[1]
    less = r[:, :, None] < r[:, None, :]
    kk = jax.lax.broadcasted_iota(jnp.int32, (1, n, n), 1)
    jj = jax.lax.broadcasted_iota(jnp.int32, (1, n, n), 2)
    tie = (r[:, :, None] == r[:, None, :]) & (kk < jj)
    rank_ref[...] = jnp.sum((less | tie).astype(jnp.int32), axis=1)


def _zeros_copy_body(cost_ref, row_ref, cost_out_ref):
    row_ref[...] = jnp.zeros_like(row_ref)
    cost_out_ref[...] = cost_ref[...]


def _sc_col_body(rank_hbm, zmat_hbm, col_hbm, buf, rank_v):
    cid = jax.lax.axis_index("c")
    sid = jax.lax.axis_index("s")
    base = (sid * _NC + cid) * _BPW
    pltpu.sync_copy(rank_hbm.at[pl.ds(base, _BPW)], rank_v)
    pltpu.sync_copy(zmat_hbm, buf)
    ones = jnp.full((_L,), 1.0, jnp.float32)
    zeros = jnp.zeros((_L,), jnp.float32)
    jbase = jax.lax.iota(jnp.int32, _L)
    for bi in range(_BPW):
        for c in range(_N // _L):
            ri = rank_v[bi, pl.ds(c * _L, _L)]
            plsc.store_scatter(buf, [ri, jbase + c * _L], ones)
        pltpu.sync_copy(buf, col_hbm.at[base + bi])
        for c in range(_N // _L):
            ri = rank_v[bi, pl.ds(c * _L, _L)]
            plsc.store_scatter(buf, [ri, jbase + c * _L], zeros)


_sc_col = functools.partial(
    pl.kernel,
    out_type=jax.ShapeDtypeStruct((_N, _N, _N), jnp.float32),
    mesh=plsc.VectorSubcoreMesh(core_axis_name="c", subcore_axis_name="s"),
    scratch_types=[
        pltpu.VMEM((_N, _N), jnp.float32),
        pltpu.VMEM((_BPW, _N), jnp.int32),
    ],
    compiler_params=pltpu.CompilerParams(needs_layout_passes=False),
)(_sc_col_body)


def kernel(cost_matrix):
    b, n, _ = cost_matrix.shape
    rkey = jax.random.fold_in(jax.random.key(0), 1)
    rand = jax.random.uniform(rkey, (b, n), dtype=jnp.float32)
    rank = pl.pallas_call(
        _rank_body,
        grid=(b // _BB,),
        in_specs=[pl.BlockSpec((_BB, n), lambda i: (i, 0))],
        out_specs=pl.BlockSpec((_BB, n), lambda i: (i, 0)),
        out_shape=jax.ShapeDtypeStruct((b, n), jnp.int32),
    )(rand)
    row_emb, cost_out = pl.pallas_call(
        _zeros_copy_body,
        grid=(b // _BB,),
        in_specs=[pl.BlockSpec((_BB, n, n), lambda i: (i, 0, 0))],
        out_specs=[
            pl.BlockSpec((_BB, n, n), lambda i: (i, 0, 0)),
            pl.BlockSpec((_BB, n, n), lambda i: (i, 0, 0)),
        ],
        out_shape=[
            jax.ShapeDtypeStruct((b, n, n), cost_matrix.dtype),
            jax.ShapeDtypeStruct((b, n, n), cost_matrix.dtype),
        ],
    )(cost_matrix)
    zmat = jnp.zeros((n, n), dtype=jnp.float32)
    col_emb = _sc_col(rank, zmat)
    return (row_emb, col_emb, cost_out)
